# baseline (speedup 1.0000x reference)
import functools

import jax
import jax.numpy as jnp
from jax import lax
from jax.experimental import pallas as pl
from jax.experimental.pallas import tpu as pltpu
from jax.experimental.pallas import tpu_sc as plsc

N = 10000
E = 320000
D_IN = 128
D_H = 64

NC = 2
NS = 16
NW = NC * NS

CH = 128
TROWS = 80
E_PAD = NW * TROWS * CH
PAD_ROWS = 128

ZCH = 640
ZLAST = N - 15 * ZCH
BCH = 80


def _mesh():
    return plsc.VectorSubcoreMesh(
        core_axis_name="c", subcore_axis_name="s", num_cores=NC, num_subcores=NS
    )


@functools.partial(
    pl.kernel,
    out_type=jax.ShapeDtypeStruct((NC, 1, N), jnp.float32),
    mesh=_mesh(),
    scratch_types=[
        pltpu.VMEM((CH,), jnp.float32),
        pltpu.VMEM((ZCH,), jnp.float32),
        pltpu.VMEM((TROWS, CH), jnp.int32),
        pltpu.VMEM_SHARED((N + PAD_ROWS,), jnp.float32),
        pltpu.SemaphoreType.DMA,
    ],
)
def _deg_kernel(dst_hbm, out_hbm, ones_v, buf_v, di_v, acc_s, sem):
    c = lax.axis_index("c")
    s = lax.axis_index("s")
    tid = c * NS + s
    row0 = tid * TROWS

    def fill_ones(i, _):
        ones_v[pl.ds(i * 16, 16)] = jnp.full((16,), 1.0, jnp.float32)
        return 0

    lax.fori_loop(0, CH // 16, fill_ones, 0)

    def fill_zero(i, _):
        buf_v[pl.ds(i * 16, 16)] = jnp.zeros((16,), jnp.float32)
        return 0

    lax.fori_loop(0, ZCH // 16, fill_zero, 0)

    @pl.when(s < NS - 1)
    def _():
        pltpu.sync_copy(buf_v, acc_s.at[pl.ds(s * ZCH, ZCH)])

    @pl.when(s == NS - 1)
    def _():
        pltpu.sync_copy(buf_v.at[pl.ds(0, ZLAST)], acc_s.at[pl.ds(15 * ZCH, ZLAST)])

    pltpu.sync_copy(dst_hbm.at[pl.ds(row0, TROWS)], di_v)

    plsc.subcore_barrier()

    def body(j, _):
        pltpu.async_copy(ones_v, acc_s.at[di_v.at[j]], sem, add=True)
        return 0

    lax.fori_loop(0, TROWS, body, 0)

    def drain(j, _):
        pltpu.make_async_copy(ones_v, acc_s.at[di_v.at[j]], sem).wait()
        return 0

    lax.fori_loop(0, TROWS, drain, 0)
    plsc.subcore_barrier()

    @pl.when(s < NS - 1)
    def _():
        pltpu.sync_copy(acc_s.at[pl.ds(s * ZCH, ZCH)], buf_v)
        pltpu.sync_copy(buf_v, out_hbm.at[c, 0, pl.ds(s * ZCH, ZCH)])

    @pl.when(s == NS - 1)
    def _():
        pltpu.sync_copy(acc_s.at[pl.ds(15 * ZCH, ZLAST)], buf_v.at[pl.ds(0, ZLAST)])
        pltpu.sync_copy(buf_v.at[pl.ds(0, ZLAST)], out_hbm.at[c, 0, pl.ds(15 * ZCH, ZLAST)])


@functools.partial(
    pl.kernel,
    out_type=jax.ShapeDtypeStruct((NC, N, D_H), jnp.float32),
    mesh=_mesh(),
    scratch_types=[
        pltpu.VMEM((TROWS, CH), jnp.int32),
        pltpu.VMEM((TROWS, CH), jnp.int32),
        [pltpu.VMEM((CH, D_H), jnp.float32)] * 4,
        pltpu.VMEM((BCH, D_H), jnp.float32),
        pltpu.VMEM_SHARED((N + PAD_ROWS, D_H), jnp.float32),
        [pltpu.SemaphoreType.DMA] * 4,
        [pltpu.SemaphoreType.DMA] * 4,
    ],
    compiler_params=pltpu.CompilerParams(use_tc_tiling_on_sc=False),
)
def _agg_kernel(
    g_hbm, src_hbm, dst_hbm, out_hbm, si_v, di_v, rows, buf_v, acc_s, gsem, ssem
):
    c = lax.axis_index("c")
    s = lax.axis_index("s")
    tid = c * NS + s
    row0 = tid * TROWS

    def fill_zero(i, _):
        def fill_col(k, _):
            buf_v[i, pl.ds(k * 16, 16)] = jnp.zeros((16,), jnp.float32)
            return 0

        lax.fori_loop(0, D_H // 16, fill_col, 0)
        return 0

    lax.fori_loop(0, BCH, fill_zero, 0)

    @pl.when(s < NS - 1)
    def _():
        def z(k, _):
            pltpu.sync_copy(buf_v, acc_s.at[pl.ds(s * ZCH + k * BCH, BCH)])
            return 0

        lax.fori_loop(0, ZCH // BCH, z, 0)

    @pl.when(s == NS - 1)
    def _():
        def z(k, _):
            pltpu.sync_copy(buf_v, acc_s.at[pl.ds(15 * ZCH + k * BCH, BCH)])
            return 0

        lax.fori_loop(0, ZLAST // BCH, z, 0)

    pltpu.sync_copy(src_hbm.at[pl.ds(row0, TROWS)], si_v)
    pltpu.sync_copy(dst_hbm.at[pl.ds(row0, TROWS)], di_v)

    plsc.subcore_barrier()

    for b in range(4):
        pltpu.async_copy(g_hbm.at[si_v.at[b]], rows[b], gsem[b])

    def body(k, _):
        j0 = 4 * k
        for b in range(4):
            j = j0 + b
            pltpu.make_async_copy(g_hbm.at[si_v.at[j]], rows[b], gsem[b]).wait()
            pltpu.async_copy(rows[b], acc_s.at[di_v.at[j]], ssem[b], add=True)
        for b in range(4):
            nx = j0 + b + 4

            @pl.when(nx < TROWS)
            def _():
                pltpu.make_async_copy(rows[b], acc_s.at[di_v.at[j0 + b]], ssem[b]).wait()
                pltpu.async_copy(g_hbm.at[si_v.at[nx]], rows[b], gsem[b])

        return 0

    lax.fori_loop(0, TROWS // 4, body, 0)

    for b in range(4):
        pltpu.make_async_copy(rows[b], acc_s.at[di_v.at[TROWS - 4 + b]], ssem[b]).wait()

    plsc.subcore_barrier()

    @pl.when(s < NS - 1)
    def _():
        def w(k, _):
            pltpu.sync_copy(acc_s.at[pl.ds(s * ZCH + k * BCH, BCH)], buf_v)
            pltpu.sync_copy(buf_v, out_hbm.at[c, pl.ds(s * ZCH + k * BCH, BCH)])
            return 0

        lax.fori_loop(0, ZCH // BCH, w, 0)

    @pl.when(s == NS - 1)
    def _():
        def w(k, _):
            pltpu.sync_copy(acc_s.at[pl.ds(15 * ZCH + k * BCH, BCH)], buf_v)
            pltpu.sync_copy(buf_v, out_hbm.at[c, pl.ds(15 * ZCH + k * BCH, BCH)])
            return 0

        lax.fori_loop(0, ZLAST // BCH, w, 0)


def _tca_body(x_ref, w1_ref, h_ref):
    h_ref[...] = jnp.dot(x_ref[...], w1_ref[...], preferred_element_type=jnp.float32)


def _tca(x, w1):
    return pl.pallas_call(
        _tca_body,
        out_shape=jax.ShapeDtypeStruct((N, D_H), jnp.float32),
    )(x, w1)


def _tcb_body(h_ref, degp_ref, g_ref, dinvb_ref):
    deg = degp_ref[0, 0, :] + degp_ref[1, 0, :] + 1.0
    dinv = lax.rsqrt(deg)
    db = jnp.broadcast_to(dinv[:, None], (N, D_H))
    dinvb_ref[...] = db
    g_ref[...] = db * h_ref[...]


def _tcb(h, degp):
    return pl.pallas_call(
        _tcb_body,
        out_shape=(
            jax.ShapeDtypeStruct((N, D_H), jnp.float32),
            jax.ShapeDtypeStruct((N, D_H), jnp.float32),
        ),
    )(h, degp)


def _tc2_body(p_ref, g_ref, dinvb_ref, b1_ref, w2_ref, g2_ref):
    db = dinvb_ref[...]
    acc = p_ref[0] + p_ref[1] + g_ref[...]
    x2 = jnp.maximum(db * acc + b1_ref[...], 0.0)
    h2 = jnp.dot(x2, w2_ref[...], preferred_element_type=jnp.float32)
    g2_ref[...] = db * h2


def _tc2(p, g, dinvb, b1, w2):
    return pl.pallas_call(
        _tc2_body,
        out_shape=jax.ShapeDtypeStruct((N, D_H), jnp.float32),
    )(p, g, dinvb, b1, w2)


def _tc3_body(p_ref, g_ref, dinvb_ref, b2_ref, wfc_ref, bfc_ref, y_ref):
    db = dinvb_ref[...]
    acc = p_ref[0] + p_ref[1] + g_ref[...]
    x3 = jnp.maximum(db * acc + b2_ref[...], 0.0)
    y_ref[...] = (
        jnp.dot(x3, wfc_ref[...], preferred_element_type=jnp.float32) + bfc_ref[...]
    )


def _tc3(p, g, dinvb, b2, wfc, bfc):
    return pl.pallas_call(
        _tc3_body,
        out_shape=jax.ShapeDtypeStruct((N, 1), jnp.float32),
    )(p, g, dinvb, b2, wfc, bfc)


def kernel(x, edge_index, W1, b1, W2, b2, Wfc, bfc):
    pad = jnp.arange(E_PAD - E, dtype=jnp.int32) % PAD_ROWS
    src2d = jnp.concatenate([edge_index[0], pad]).reshape(E_PAD // CH, CH)
    dst2d = jnp.concatenate([edge_index[1], pad + N]).reshape(E_PAD // CH, CH)

    degp = _deg_kernel(dst2d)
    h1 = _tca(x, W1)
    g1, dinvb = _tcb(h1, degp)
    p1 = _agg_kernel(g1, src2d, dst2d)
    g2 = _tc2(p1, g1, dinvb, b1.reshape(1, D_H), W2)
    p2 = _agg_kernel(g2, src2d, dst2d)
    return _tc3(p2, g2, dinvb, b2.reshape(1, D_H), Wfc, bfc.reshape(1, 1))


# speedup vs baseline: 1.0372x; 1.0372x over previous
import functools

import jax
import jax.numpy as jnp
from jax import lax
from jax.experimental import pallas as pl
from jax.experimental.pallas import tpu as pltpu
from jax.experimental.pallas import tpu_sc as plsc

N = 10000
E = 320000
D_IN = 128
D_H = 64

NC = 2
NS = 16
NW = NC * NS

CH = 128
TROWS = 80
E_PAD = NW * TROWS * CH
PAD_ROWS = 128

ZCH = 640
ZLAST = N - 15 * ZCH
BCH = 80


def _mesh():
    return plsc.VectorSubcoreMesh(
        core_axis_name="c", subcore_axis_name="s", num_cores=NC, num_subcores=NS
    )


@functools.partial(
    pl.kernel,
    out_type=jax.ShapeDtypeStruct((NC, 1, N), jnp.float32),
    mesh=_mesh(),
    scratch_types=[
        pltpu.VMEM((CH,), jnp.float32),
        pltpu.VMEM((ZCH,), jnp.float32),
        pltpu.VMEM((TROWS, CH), jnp.int32),
        pltpu.VMEM_SHARED((N + PAD_ROWS,), jnp.float32),
        pltpu.SemaphoreType.DMA,
    ],
)
def _deg_kernel(dst_hbm, out_hbm, ones_v, buf_v, di_v, acc_s, sem):
    c = lax.axis_index("c")
    s = lax.axis_index("s")
    tid = c * NS + s
    row0 = tid * TROWS

    def fill_ones(i, _):
        ones_v[pl.ds(i * 16, 16)] = jnp.full((16,), 1.0, jnp.float32)
        return 0

    lax.fori_loop(0, CH // 16, fill_ones, 0)

    def fill_zero(i, _):
        buf_v[pl.ds(i * 16, 16)] = jnp.zeros((16,), jnp.float32)
        return 0

    lax.fori_loop(0, ZCH // 16, fill_zero, 0)

    @pl.when(s < NS - 1)
    def _():
        pltpu.sync_copy(buf_v, acc_s.at[pl.ds(s * ZCH, ZCH)])

    @pl.when(s == NS - 1)
    def _():
        pltpu.sync_copy(buf_v.at[pl.ds(0, ZLAST)], acc_s.at[pl.ds(15 * ZCH, ZLAST)])

    pltpu.sync_copy(dst_hbm.at[pl.ds(row0, TROWS)], di_v)

    plsc.subcore_barrier()

    def body(j, _):
        pltpu.async_copy(ones_v, acc_s.at[di_v.at[j]], sem, add=True)
        return 0

    lax.fori_loop(0, TROWS, body, 0)

    def drain(j, _):
        pltpu.make_async_copy(ones_v, acc_s.at[di_v.at[j]], sem).wait()
        return 0

    lax.fori_loop(0, TROWS, drain, 0)
    plsc.subcore_barrier()

    @pl.when(s < NS - 1)
    def _():
        pltpu.sync_copy(acc_s.at[pl.ds(s * ZCH, ZCH)], buf_v)
        pltpu.sync_copy(buf_v, out_hbm.at[c, 0, pl.ds(s * ZCH, ZCH)])

    @pl.when(s == NS - 1)
    def _():
        pltpu.sync_copy(acc_s.at[pl.ds(15 * ZCH, ZLAST)], buf_v.at[pl.ds(0, ZLAST)])
        pltpu.sync_copy(buf_v.at[pl.ds(0, ZLAST)], out_hbm.at[c, 0, pl.ds(15 * ZCH, ZLAST)])


@functools.partial(
    pl.kernel,
    out_type=jax.ShapeDtypeStruct((NC, N, D_H), jnp.float32),
    mesh=_mesh(),
    scratch_types=[
        pltpu.VMEM((TROWS, CH), jnp.int32),
        pltpu.VMEM((TROWS, CH), jnp.int32),
        [pltpu.VMEM((CH, D_H), jnp.float32)] * 4,
        pltpu.VMEM((BCH, D_H), jnp.float32),
        pltpu.VMEM_SHARED((N + PAD_ROWS, D_H), jnp.float32),
        [pltpu.SemaphoreType.DMA] * 4,
        [pltpu.SemaphoreType.DMA] * 4,
        pltpu.SemaphoreType.DMA,
    ],
    compiler_params=pltpu.CompilerParams(use_tc_tiling_on_sc=False),
)
def _agg_kernel(
    g_hbm, src_hbm, dst_hbm, out_hbm, si_v, di_v, rows, buf_v, acc_s, gsem, ssem, psem
):
    c = lax.axis_index("c")
    s = lax.axis_index("s")
    tid = c * NS + s
    row0 = tid * TROWS

    def fill_zero(i, _):
        def fill_col(k, _):
            buf_v[i, pl.ds(k * 16, 16)] = jnp.zeros((16,), jnp.float32)
            return 0

        lax.fori_loop(0, D_H // 16, fill_col, 0)
        return 0

    lax.fori_loop(0, BCH, fill_zero, 0)

    pltpu.async_copy(src_hbm.at[pl.ds(row0, TROWS)], si_v, psem)
    pltpu.async_copy(dst_hbm.at[pl.ds(row0, TROWS)], di_v, psem)

    @pl.when(s < NS - 1)
    def _():
        def z(k, _):
            pltpu.sync_copy(buf_v, acc_s.at[pl.ds(s * ZCH + k * BCH, BCH)])
            return 0

        lax.fori_loop(0, ZCH // BCH, z, 0)

    @pl.when(s == NS - 1)
    def _():
        def z(k, _):
            pltpu.sync_copy(buf_v, acc_s.at[pl.ds(15 * ZCH + k * BCH, BCH)])
            return 0

        lax.fori_loop(0, ZLAST // BCH, z, 0)

    pltpu.make_async_copy(src_hbm.at[pl.ds(row0, TROWS)], si_v, psem).wait()
    pltpu.make_async_copy(dst_hbm.at[pl.ds(row0, TROWS)], di_v, psem).wait()

    plsc.subcore_barrier()

    for b in range(4):
        pltpu.async_copy(g_hbm.at[si_v.at[b]], rows[b], gsem[b])

    def body(k, _):
        j0 = 4 * k
        for b in range(4):
            j = j0 + b
            pltpu.make_async_copy(g_hbm.at[si_v.at[j]], rows[b], gsem[b]).wait()
            pltpu.async_copy(rows[b], acc_s.at[di_v.at[j]], ssem[b], add=True)
        for b in range(4):
            nx = j0 + b + 4

            @pl.when(nx < TROWS)
            def _():
                pltpu.make_async_copy(rows[b], acc_s.at[di_v.at[j0 + b]], ssem[b]).wait()
                pltpu.async_copy(g_hbm.at[si_v.at[nx]], rows[b], gsem[b])

        return 0

    lax.fori_loop(0, TROWS // 4, body, 0)

    for b in range(4):
        pltpu.make_async_copy(rows[b], acc_s.at[di_v.at[TROWS - 4 + b]], ssem[b]).wait()

    plsc.subcore_barrier()

    @pl.when(s < NS - 1)
    def _():
        for k in range(ZCH // CH):
            b = rows[k % 2]
            if k >= 2:
                pltpu.make_async_copy(
                    b, out_hbm.at[c, pl.ds(s * ZCH + (k - 2) * CH, CH)], gsem[k % 2]
                ).wait()
            pltpu.sync_copy(acc_s.at[pl.ds(s * ZCH + k * CH, CH)], b)
            pltpu.async_copy(b, out_hbm.at[c, pl.ds(s * ZCH + k * CH, CH)], gsem[k % 2])
        for k in range(ZCH // CH - 2, ZCH // CH):
            pltpu.make_async_copy(
                rows[k % 2], out_hbm.at[c, pl.ds(s * ZCH + k * CH, CH)], gsem[k % 2]
            ).wait()

    @pl.when(s == NS - 1)
    def _():
        for k in range(3):
            b = rows[k % 2]
            if k >= 2:
                pltpu.make_async_copy(
                    b, out_hbm.at[c, pl.ds(15 * ZCH + (k - 2) * CH, CH)], gsem[k % 2]
                ).wait()
            pltpu.sync_copy(acc_s.at[pl.ds(15 * ZCH + k * CH, CH)], b)
            pltpu.async_copy(b, out_hbm.at[c, pl.ds(15 * ZCH + k * CH, CH)], gsem[k % 2])
        pltpu.sync_copy(
            acc_s.at[pl.ds(15 * ZCH + 3 * CH, ZLAST - 3 * CH)],
            rows[2].at[pl.ds(0, ZLAST - 3 * CH)],
        )
        pltpu.async_copy(
            rows[2].at[pl.ds(0, ZLAST - 3 * CH)],
            out_hbm.at[c, pl.ds(15 * ZCH + 3 * CH, ZLAST - 3 * CH)],
            gsem[2],
        )
        for k in range(1, 3):
            pltpu.make_async_copy(
                rows[k % 2], out_hbm.at[c, pl.ds(15 * ZCH + k * CH, CH)], gsem[k % 2]
            ).wait()
        pltpu.make_async_copy(
            rows[2].at[pl.ds(0, ZLAST - 3 * CH)],
            out_hbm.at[c, pl.ds(15 * ZCH + 3 * CH, ZLAST - 3 * CH)],
            gsem[2],
        ).wait()


def _tc1_body(x_ref, w1_ref, degp_ref, g_ref, dinvb_ref):
    deg = degp_ref[0, 0, :] + degp_ref[1, 0, :] + 1.0
    dinv = lax.rsqrt(deg)
    db = jnp.broadcast_to(dinv[:, None], (N, D_H))
    h = jnp.dot(x_ref[...], w1_ref[...], preferred_element_type=jnp.float32)
    dinvb_ref[...] = db
    g_ref[...] = db * h


def _tc1(x, w1, degp):
    return pl.pallas_call(
        _tc1_body,
        out_shape=(
            jax.ShapeDtypeStruct((N, D_H), jnp.float32),
            jax.ShapeDtypeStruct((N, D_H), jnp.float32),
        ),
    )(x, w1, degp)


def _tc2_body(p_ref, g_ref, dinvb_ref, b1_ref, w2_ref, g2_ref):
    db = dinvb_ref[...]
    acc = p_ref[0] + p_ref[1] + g_ref[...]
    x2 = jnp.maximum(db * acc + b1_ref[...], 0.0)
    h2 = jnp.dot(x2, w2_ref[...], preferred_element_type=jnp.float32)
    g2_ref[...] = db * h2


def _tc2(p, g, dinvb, b1, w2):
    return pl.pallas_call(
        _tc2_body,
        out_shape=jax.ShapeDtypeStruct((N, D_H), jnp.float32),
    )(p, g, dinvb, b1, w2)


def _tc3_body(p_ref, g_ref, dinvb_ref, b2_ref, wfc_ref, bfc_ref, y_ref):
    db = dinvb_ref[...]
    acc = p_ref[0] + p_ref[1] + g_ref[...]
    x3 = jnp.maximum(db * acc + b2_ref[...], 0.0)
    y_ref[...] = (
        jnp.dot(x3, wfc_ref[...], preferred_element_type=jnp.float32) + bfc_ref[...]
    )


def _tc3(p, g, dinvb, b2, wfc, bfc):
    return pl.pallas_call(
        _tc3_body,
        out_shape=jax.ShapeDtypeStruct((N, 1), jnp.float32),
    )(p, g, dinvb, b2, wfc, bfc)


def kernel(x, edge_index, W1, b1, W2, b2, Wfc, bfc):
    pad = jnp.arange(E_PAD - E, dtype=jnp.int32) % PAD_ROWS
    src2d = jnp.concatenate([edge_index[0], pad]).reshape(E_PAD // CH, CH)
    dst2d = jnp.concatenate([edge_index[1], pad + N]).reshape(E_PAD // CH, CH)

    degp = _deg_kernel(dst2d)
    g1, dinvb = _tc1(x, W1, degp)
    p1 = _agg_kernel(g1, src2d, dst2d)
    g2 = _tc2(p1, g1, dinvb, b1.reshape(1, D_H), W2)
    p2 = _agg_kernel(g2, src2d, dst2d)
    return _tc3(p2, g2, dinvb, b2.reshape(1, D_H), Wfc, bfc.reshape(1, 1))


# speedup vs baseline: 1.0391x; 1.0018x over previous
import functools

import jax
import jax.numpy as jnp
from jax import lax
from jax.experimental import pallas as pl
from jax.experimental.pallas import tpu as pltpu
from jax.experimental.pallas import tpu_sc as plsc

N = 10000
E = 320000
D_IN = 128
D_H = 64

NC = 2
NS = 16
NW = NC * NS

CH = 128
TROWS = 80
E_PAD = NW * TROWS * CH
PAD_ROWS = 128

ZCH = 640
ZLAST = N - 15 * ZCH
BCH = 80


def _mesh():
    return plsc.VectorSubcoreMesh(
        core_axis_name="c", subcore_axis_name="s", num_cores=NC, num_subcores=NS
    )


@functools.partial(
    pl.kernel,
    out_type=jax.ShapeDtypeStruct((NC, 1, N), jnp.float32),
    mesh=_mesh(),
    scratch_types=[
        pltpu.VMEM((CH,), jnp.float32),
        pltpu.VMEM((ZCH,), jnp.float32),
        pltpu.VMEM((TROWS, CH), jnp.int32),
        pltpu.VMEM_SHARED((N + PAD_ROWS,), jnp.float32),
        pltpu.SemaphoreType.DMA,
    ],
    compiler_params=pltpu.CompilerParams(skip_device_barrier=True),
)
def _deg_kernel(dst_hbm, out_hbm, ones_v, buf_v, di_v, acc_s, sem):
    c = lax.axis_index("c")
    s = lax.axis_index("s")
    tid = c * NS + s
    row0 = tid * TROWS

    def fill_ones(i, _):
        ones_v[pl.ds(i * 16, 16)] = jnp.full((16,), 1.0, jnp.float32)
        return 0

    lax.fori_loop(0, CH // 16, fill_ones, 0)

    def fill_zero(i, _):
        buf_v[pl.ds(i * 16, 16)] = jnp.zeros((16,), jnp.float32)
        return 0

    lax.fori_loop(0, ZCH // 16, fill_zero, 0)

    @pl.when(s < NS - 1)
    def _():
        pltpu.sync_copy(buf_v, acc_s.at[pl.ds(s * ZCH, ZCH)])

    @pl.when(s == NS - 1)
    def _():
        pltpu.sync_copy(buf_v.at[pl.ds(0, ZLAST)], acc_s.at[pl.ds(15 * ZCH, ZLAST)])

    pltpu.sync_copy(dst_hbm.at[pl.ds(row0, TROWS)], di_v)

    plsc.subcore_barrier()

    def body(j, _):
        pltpu.async_copy(ones_v, acc_s.at[di_v.at[j]], sem, add=True)
        return 0

    lax.fori_loop(0, TROWS, body, 0)

    def drain(j, _):
        pltpu.make_async_copy(ones_v, acc_s.at[di_v.at[j]], sem).wait()
        return 0

    lax.fori_loop(0, TROWS, drain, 0)
    plsc.subcore_barrier()

    @pl.when(s < NS - 1)
    def _():
        pltpu.sync_copy(acc_s.at[pl.ds(s * ZCH, ZCH)], buf_v)
        pltpu.sync_copy(buf_v, out_hbm.at[c, 0, pl.ds(s * ZCH, ZCH)])

    @pl.when(s == NS - 1)
    def _():
        pltpu.sync_copy(acc_s.at[pl.ds(15 * ZCH, ZLAST)], buf_v.at[pl.ds(0, ZLAST)])
        pltpu.sync_copy(buf_v.at[pl.ds(0, ZLAST)], out_hbm.at[c, 0, pl.ds(15 * ZCH, ZLAST)])


@functools.partial(
    pl.kernel,
    out_type=jax.ShapeDtypeStruct((NC, N, D_H), jnp.float32),
    mesh=_mesh(),
    scratch_types=[
        pltpu.VMEM((TROWS, CH), jnp.int32),
        pltpu.VMEM((TROWS, CH), jnp.int32),
        [pltpu.VMEM((CH, D_H), jnp.float32)] * 4,
        pltpu.VMEM((BCH, D_H), jnp.float32),
        pltpu.VMEM_SHARED((N + PAD_ROWS, D_H), jnp.float32),
        [pltpu.SemaphoreType.DMA] * 4,
        [pltpu.SemaphoreType.DMA] * 4,
        pltpu.SemaphoreType.DMA,
    ],
    compiler_params=pltpu.CompilerParams(use_tc_tiling_on_sc=False, skip_device_barrier=True),
)
def _agg_kernel(
    g_hbm, src_hbm, dst_hbm, out_hbm, si_v, di_v, rows, buf_v, acc_s, gsem, ssem, psem
):
    c = lax.axis_index("c")
    s = lax.axis_index("s")
    tid = c * NS + s
    row0 = tid * TROWS

    def fill_zero(i, _):
        def fill_col(k, _):
            buf_v[i, pl.ds(k * 16, 16)] = jnp.zeros((16,), jnp.float32)
            return 0

        lax.fori_loop(0, D_H // 16, fill_col, 0)
        return 0

    lax.fori_loop(0, BCH, fill_zero, 0)

    pltpu.async_copy(src_hbm.at[pl.ds(row0, TROWS)], si_v, psem)
    pltpu.async_copy(dst_hbm.at[pl.ds(row0, TROWS)], di_v, psem)

    @pl.when(s < NS - 1)
    def _():
        def z(k, _):
            pltpu.sync_copy(buf_v, acc_s.at[pl.ds(s * ZCH + k * BCH, BCH)])
            return 0

        lax.fori_loop(0, ZCH // BCH, z, 0)

    @pl.when(s == NS - 1)
    def _():
        def z(k, _):
            pltpu.sync_copy(buf_v, acc_s.at[pl.ds(15 * ZCH + k * BCH, BCH)])
            return 0

        lax.fori_loop(0, ZLAST // BCH, z, 0)

    pltpu.make_async_copy(src_hbm.at[pl.ds(row0, TROWS)], si_v, psem).wait()
    pltpu.make_async_copy(dst_hbm.at[pl.ds(row0, TROWS)], di_v, psem).wait()

    plsc.subcore_barrier()

    for b in range(4):
        pltpu.async_copy(g_hbm.at[si_v.at[b]], rows[b], gsem[b])

    def body(k, _):
        j0 = 4 * k
        for b in range(4):
            j = j0 + b
            pltpu.make_async_copy(g_hbm.at[si_v.at[j]], rows[b], gsem[b]).wait()
            pltpu.async_copy(rows[b], acc_s.at[di_v.at[j]], ssem[b], add=True)
        for b in range(4):
            nx = j0 + b + 4

            @pl.when(nx < TROWS)
            def _():
                pltpu.make_async_copy(rows[b], acc_s.at[di_v.at[j0 + b]], ssem[b]).wait()
                pltpu.async_copy(g_hbm.at[si_v.at[nx]], rows[b], gsem[b])

        return 0

    lax.fori_loop(0, TROWS // 4, body, 0)

    for b in range(4):
        pltpu.make_async_copy(rows[b], acc_s.at[di_v.at[TROWS - 4 + b]], ssem[b]).wait()

    plsc.subcore_barrier()

    @pl.when(s < NS - 1)
    def _():
        for k in range(ZCH // CH):
            b = rows[k % 2]
            if k >= 2:
                pltpu.make_async_copy(
                    b, out_hbm.at[c, pl.ds(s * ZCH + (k - 2) * CH, CH)], gsem[k % 2]
                ).wait()
            pltpu.sync_copy(acc_s.at[pl.ds(s * ZCH + k * CH, CH)], b)
            pltpu.async_copy(b, out_hbm.at[c, pl.ds(s * ZCH + k * CH, CH)], gsem[k % 2])
        for k in range(ZCH // CH - 2, ZCH // CH):
            pltpu.make_async_copy(
                rows[k % 2], out_hbm.at[c, pl.ds(s * ZCH + k * CH, CH)], gsem[k % 2]
            ).wait()

    @pl.when(s == NS - 1)
    def _():
        for k in range(3):
            b = rows[k % 2]
            if k >= 2:
                pltpu.make_async_copy(
                    b, out_hbm.at[c, pl.ds(15 * ZCH + (k - 2) * CH, CH)], gsem[k % 2]
                ).wait()
            pltpu.sync_copy(acc_s.at[pl.ds(15 * ZCH + k * CH, CH)], b)
            pltpu.async_copy(b, out_hbm.at[c, pl.ds(15 * ZCH + k * CH, CH)], gsem[k % 2])
        pltpu.sync_copy(
            acc_s.at[pl.ds(15 * ZCH + 3 * CH, ZLAST - 3 * CH)],
            rows[2].at[pl.ds(0, ZLAST - 3 * CH)],
        )
        pltpu.async_copy(
            rows[2].at[pl.ds(0, ZLAST - 3 * CH)],
            out_hbm.at[c, pl.ds(15 * ZCH + 3 * CH, ZLAST - 3 * CH)],
            gsem[2],
        )
        for k in range(1, 3):
            pltpu.make_async_copy(
                rows[k % 2], out_hbm.at[c, pl.ds(15 * ZCH + k * CH, CH)], gsem[k % 2]
            ).wait()
        pltpu.make_async_copy(
            rows[2].at[pl.ds(0, ZLAST - 3 * CH)],
            out_hbm.at[c, pl.ds(15 * ZCH + 3 * CH, ZLAST - 3 * CH)],
            gsem[2],
        ).wait()


def _tc1_body(x_ref, w1_ref, degp_ref, g_ref, dinvb_ref):
    deg = degp_ref[0, 0, :] + degp_ref[1, 0, :] + 1.0
    dinv = lax.rsqrt(deg)
    db = jnp.broadcast_to(dinv[:, None], (N, D_H))
    h = jnp.dot(x_ref[...], w1_ref[...], preferred_element_type=jnp.float32)
    dinvb_ref[...] = db
    g_ref[...] = db * h


def _tc1(x, w1, degp):
    return pl.pallas_call(
        _tc1_body,
        out_shape=(
            jax.ShapeDtypeStruct((N, D_H), jnp.float32),
            jax.ShapeDtypeStruct((N, D_H), jnp.float32),
        ),
    )(x, w1, degp)


def _tc2_body(p_ref, g_ref, dinvb_ref, b1_ref, w2_ref, g2_ref):
    db = dinvb_ref[...]
    acc = p_ref[0] + p_ref[1] + g_ref[...]
    x2 = jnp.maximum(db * acc + b1_ref[...], 0.0)
    h2 = jnp.dot(x2, w2_ref[...], preferred_element_type=jnp.float32)
    g2_ref[...] = db * h2


def _tc2(p, g, dinvb, b1, w2):
    return pl.pallas_call(
        _tc2_body,
        out_shape=jax.ShapeDtypeStruct((N, D_H), jnp.float32),
    )(p, g, dinvb, b1, w2)


def _tc3_body(p_ref, g_ref, dinvb_ref, b2_ref, wfc_ref, bfc_ref, y_ref):
    db = dinvb_ref[...]
    acc = p_ref[0] + p_ref[1] + g_ref[...]
    x3 = jnp.maximum(db * acc + b2_ref[...], 0.0)
    y_ref[...] = (
        jnp.dot(x3, wfc_ref[...], preferred_element_type=jnp.float32) + bfc_ref[...]
    )


def _tc3(p, g, dinvb, b2, wfc, bfc):
    return pl.pallas_call(
        _tc3_body,
        out_shape=jax.ShapeDtypeStruct((N, 1), jnp.float32),
    )(p, g, dinvb, b2, wfc, bfc)


def kernel(x, edge_index, W1, b1, W2, b2, Wfc, bfc):
    pad = jnp.arange(E_PAD - E, dtype=jnp.int32) % PAD_ROWS
    src2d = jnp.concatenate([edge_index[0], pad]).reshape(E_PAD // CH, CH)
    dst2d = jnp.concatenate([edge_index[1], pad + N]).reshape(E_PAD // CH, CH)

    degp = _deg_kernel(dst2d)
    g1, dinvb = _tc1(x, W1, degp)
    p1 = _agg_kernel(g1, src2d, dst2d)
    g2 = _tc2(p1, g1, dinvb, b1.reshape(1, D_H), W2)
    p2 = _agg_kernel(g2, src2d, dst2d)
    return _tc3(p2, g2, dinvb, b2.reshape(1, D_H), Wfc, bfc.reshape(1, 1))
